# baseline (speedup 1.0000x reference)
import functools

import jax
import jax.numpy as jnp
from jax import lax
from jax.experimental import pallas as pl
from jax.experimental.pallas import tpu as pltpu
from jax.experimental.pallas import tpu_sc as plsc

BATCH = 16384
EMBED_DIM = 64
AVG_RATING = 3.0

_NC = 2
_NS = 16
_NW = _NC * _NS
_BPW = BATCH // _NW
_CHUNK = 128
_NCHUNK = _BPW // _CHUNK
_GROUPS = _BPW // 16
_COLS = EMBED_DIM * _BPW


def _body(user_hbm, item_hbm, ut_hbm, it_hbm,
          user_bias_hbm, item_bias_hbm,
          out_hbm, ub_hbm, ib_hbm,
          idx_u, idx_i, ucols, icols, ub_v, ib_v, out_v,
          sem_u, sem_i, semb):
    wid = lax.axis_index("s") * _NC + lax.axis_index("c")
    base = wid * _BPW

    for k in range(_NCHUNK):
        pltpu.sync_copy(user_hbm.at[pl.ds(base + k * _CHUNK, _CHUNK)], idx_u.at[k])
        pltpu.sync_copy(item_hbm.at[pl.ds(base + k * _CHUNK, _CHUNK)], idx_i.at[k])

    bias_copies = []
    for k in range(_NCHUNK):
        sl = pl.ds(k * _CHUNK, _CHUNK)
        bias_copies.append(pltpu.async_copy(user_bias_hbm.at[idx_u.at[k]], ub_v.at[sl], semb))
        bias_copies.append(pltpu.async_copy(item_bias_hbm.at[idx_i.at[k]], ib_v.at[sl], semb))

    def fire(c, carry):
        for k in range(_NCHUNK):
            pltpu.async_copy(ut_hbm.at[c].at[idx_u.at[k]],
                             ucols.at[pl.ds(c * _BPW + k * _CHUNK, _CHUNK)], sem_u)
            pltpu.async_copy(it_hbm.at[c].at[idx_i.at[k]],
                             icols.at[pl.ds(c * _BPW + k * _CHUNK, _CHUNK)], sem_i)
        return carry

    lax.fori_loop(0, EMBED_DIM, fire, 0)

    pltpu.make_async_copy(ut_hbm.at[0].at[pl.ds(0, _COLS)], ucols, sem_u).wait()
    pltpu.make_async_copy(it_hbm.at[0].at[pl.ds(0, _COLS)], icols, sem_i).wait()
    for c in bias_copies:
        c.wait()

    def group(g, carry):
        gsl = pl.ds(g * 16, 16)
        acc = None
        for c in range(EMBED_DIM):
            u = ucols[pl.ds(c * _BPW + g * 16, 16)]
            v = icols[pl.ds(c * _BPW + g * 16, 16)]
            acc = u * v if acc is None else acc + u * v
        out_v[gsl] = AVG_RATING + ub_v[gsl] + ib_v[gsl] + acc
        return carry

    lax.fori_loop(0, _GROUPS, group, 0)

    pltpu.sync_copy(out_v, out_hbm.at[pl.ds(base, _BPW)])
    pltpu.sync_copy(ub_v, ub_hbm.at[pl.ds(base, _BPW)])
    pltpu.sync_copy(ib_v, ib_hbm.at[pl.ds(base, _BPW)])


@functools.partial(
    pl.kernel,
    mesh=plsc.VectorSubcoreMesh(core_axis_name="c", subcore_axis_name="s"),
    compiler_params=pltpu.CompilerParams(use_tc_tiling_on_sc=False),
    out_type=(
        jax.ShapeDtypeStruct((BATCH,), jnp.float32),
        jax.ShapeDtypeStruct((BATCH,), jnp.float32),
        jax.ShapeDtypeStruct((BATCH,), jnp.float32),
    ),
    scratch_types=[
        pltpu.VMEM((_NCHUNK, _CHUNK), jnp.int32),
        pltpu.VMEM((_NCHUNK, _CHUNK), jnp.int32),
        pltpu.VMEM((_COLS,), jnp.float32),
        pltpu.VMEM((_COLS,), jnp.float32),
        pltpu.VMEM((_BPW,), jnp.float32),
        pltpu.VMEM((_BPW,), jnp.float32),
        pltpu.VMEM((_BPW,), jnp.float32),
        pltpu.SemaphoreType.DMA,
        pltpu.SemaphoreType.DMA,
        pltpu.SemaphoreType.DMA,
    ],
)
def _svd_sc(*refs):
    _body(*refs)


def kernel(user, item, user_emb, item_emb, user_bias, item_bias):
    return _svd_sc(user, item, user_emb.T, item_emb.T, user_bias, item_bias)


# speedup vs baseline: 11.4995x; 11.4995x over previous
import functools

import jax
import jax.numpy as jnp
from jax import lax
from jax.experimental import pallas as pl
from jax.experimental.pallas import tpu as pltpu
from jax.experimental.pallas import tpu_sc as plsc

BATCH = 16384
EMBED_DIM = 64
AVG_RATING = 3.0

_NC = 2
_NS = 16
_NW = _NC * _NS
_BPW = BATCH // _NW
_CHUNK = 128
_NCHUNK = _BPW // _CHUNK
_CROWS = 16
_NCC = _BPW // _CROWS
_HALF = _CROWS * 8

_LANE16 = None


def _body(user_hbm, item_hbm, ue_hbm, ie_hbm,
          user_bias_hbm, item_bias_hbm,
          out_hbm, ub_hbm, ib_hbm,
          idx_u, idx_i, idx_uf, idx_if, slabs_u, slabs_i,
          ub_v, ib_v, out_v, sem_u, sem_i, semb):
    wid = lax.axis_index("s") * _NC + lax.axis_index("c")
    base = wid * _BPW

    for k in range(_NCHUNK):
        src = user_hbm.at[pl.ds(base + k * _CHUNK, _CHUNK)]
        pltpu.sync_copy(src, idx_u.at[k])
        pltpu.sync_copy(src, idx_uf.at[pl.ds(k * _CHUNK, _CHUNK)])
        src = item_hbm.at[pl.ds(base + k * _CHUNK, _CHUNK)]
        pltpu.sync_copy(src, idx_i.at[k])
        pltpu.sync_copy(src, idx_if.at[pl.ds(k * _CHUNK, _CHUNK)])

    bias_copies = []
    for k in range(_NCHUNK):
        sl = pl.ds(k * _CHUNK, _CHUNK)
        bias_copies.append(pltpu.async_copy(user_bias_hbm.at[idx_u.at[k]], ub_v.at[sl], semb))
        bias_copies.append(pltpu.async_copy(item_bias_hbm.at[idx_i.at[k]], ib_v.at[sl], semb))

    lane = lax.iota(jnp.int32, 16)

    def fire_chunk(c):
        par = (c & 1) * _HALF
        iu = idx_uf[pl.ds(c * _CROWS, _CROWS)]
        ii = idx_if[pl.ds(c * _CROWS, _CROWS)]
        au = iu & (-8)
        ai = ii & (-8)
        for j in range(_CROWS):
            dsl = pl.ds(par + j * 8, 8)
            su = pl.multiple_of(au[j], 8)
            si = pl.multiple_of(ai[j], 8)
            pltpu.async_copy(ue_hbm.at[pl.ds(su, 8)], slabs_u.at[dsl], sem_u)
            pltpu.async_copy(ie_hbm.at[pl.ds(si, 8)], slabs_i.at[dsl], sem_i)

    def compute_chunk(c):
        par = (c & 1) * _HALF
        pltpu.make_async_copy(ue_hbm.at[pl.ds(0, _HALF)],
                              slabs_u.at[pl.ds(par, _HALF)], sem_u).wait()
        pltpu.make_async_copy(ie_hbm.at[pl.ds(0, _HALF)],
                              slabs_i.at[pl.ds(par, _HALF)], sem_i).wait()
        iu = idx_uf[pl.ds(c * _CROWS, _CROWS)]
        ii = idx_if[pl.ds(c * _CROWS, _CROWS)]
        r8u = iu & 7
        r8i = ii & 7
        vecs = []
        for j in range(_CROWS):
            ru = par + j * 8 + r8u[j]
            ri = par + j * 8 + r8i[j]
            acc = None
            for t in range(EMBED_DIM // 16):
                tsl = pl.ds(t * 16, 16)
                uv = slabs_u[ru, tsl]
                iv = slabs_i[ri, tsl]
                acc = uv * iv if acc is None else acc + uv * iv
            vecs.append(acc)
        sh = 1
        while len(vecs) > 1:
            idxs = lane ^ sh
            m = (lane & sh) != 0
            nxt = []
            for q in range(len(vecs) // 2):
                u, v = vecs[2 * q], vecs[2 * q + 1]
                gu = u.at[idxs].get(mode="promise_in_bounds")
                gv = v.at[idxs].get(mode="promise_in_bounds")
                nxt.append(jnp.where(m, v + gv, u + gu))
            vecs = nxt
            sh *= 2
        osl = pl.ds(c * _CROWS, _CROWS)
        out_v[osl] = AVG_RATING + ub_v[osl] + ib_v[osl] + vecs[0]

    def step(c, carry):
        fire_chunk(c)

        @pl.when(c > 0)
        def _():
            compute_chunk(c - 1)

        return carry

    lax.fori_loop(0, _NCC, step, 0)
    for cp in bias_copies:
        cp.wait()
    compute_chunk(_NCC - 1)

    pltpu.sync_copy(out_v, out_hbm.at[pl.ds(base, _BPW)])
    pltpu.sync_copy(ub_v, ub_hbm.at[pl.ds(base, _BPW)])
    pltpu.sync_copy(ib_v, ib_hbm.at[pl.ds(base, _BPW)])


@functools.partial(
    pl.kernel,
    mesh=plsc.VectorSubcoreMesh(core_axis_name="c", subcore_axis_name="s"),
    out_type=(
        jax.ShapeDtypeStruct((BATCH,), jnp.float32),
        jax.ShapeDtypeStruct((BATCH,), jnp.float32),
        jax.ShapeDtypeStruct((BATCH,), jnp.float32),
    ),
    scratch_types=[
        pltpu.VMEM((_NCHUNK, _CHUNK), jnp.int32),
        pltpu.VMEM((_NCHUNK, _CHUNK), jnp.int32),
        pltpu.VMEM((_BPW,), jnp.int32),
        pltpu.VMEM((_BPW,), jnp.int32),
        pltpu.VMEM((2 * _HALF, EMBED_DIM), jnp.float32),
        pltpu.VMEM((2 * _HALF, EMBED_DIM), jnp.float32),
        pltpu.VMEM((_BPW,), jnp.float32),
        pltpu.VMEM((_BPW,), jnp.float32),
        pltpu.VMEM((_BPW,), jnp.float32),
        pltpu.SemaphoreType.DMA,
        pltpu.SemaphoreType.DMA,
        pltpu.SemaphoreType.DMA,
    ],
)
def _svd_sc(*refs):
    _body(*refs)


def kernel(user, item, user_emb, item_emb, user_bias, item_bias):
    return _svd_sc(user, item, user_emb, item_emb, user_bias, item_bias)


# speedup vs baseline: 16.2933x; 1.4169x over previous
import functools

import jax
import jax.numpy as jnp
from jax import lax
from jax.experimental import pallas as pl
from jax.experimental.pallas import tpu as pltpu
from jax.experimental.pallas import tpu_sc as plsc

BATCH = 16384
EMBED_DIM = 64
AVG_RATING = 3.0

_NC = 2
_NS = 16
_NW = _NC * _NS
_BPW = BATCH // _NW
_CHUNK = 128
_NCHUNK = _BPW // _CHUNK
_CROWS = 16
_NCC = _BPW // _CROWS
_HALF = _CROWS * 8

_LANE16 = None


def _body(user_hbm, item_hbm, ue_hbm, ie_hbm,
          user_bias_hbm, item_bias_hbm,
          out_hbm, ub_hbm, ib_hbm,
          idx_u, idx_i, idx_uf, idx_if, slabs_u, slabs_i,
          ub_v, ib_v, out_v, sem_u, sem_i, semb):
    wid = lax.axis_index("s") * _NC + lax.axis_index("c")
    base = wid * _BPW

    for k in range(_NCHUNK):
        src = user_hbm.at[pl.ds(base + k * _CHUNK, _CHUNK)]
        pltpu.sync_copy(src, idx_u.at[k])
        pltpu.sync_copy(src, idx_uf.at[pl.ds(k * _CHUNK, _CHUNK)])
        src = item_hbm.at[pl.ds(base + k * _CHUNK, _CHUNK)]
        pltpu.sync_copy(src, idx_i.at[k])
        pltpu.sync_copy(src, idx_if.at[pl.ds(k * _CHUNK, _CHUNK)])

    bias_copies = []
    for k in range(_NCHUNK):
        sl = pl.ds(k * _CHUNK, _CHUNK)
        bias_copies.append(pltpu.async_copy(user_bias_hbm.at[idx_u.at[k]], ub_v.at[sl], semb))
        bias_copies.append(pltpu.async_copy(item_bias_hbm.at[idx_i.at[k]], ib_v.at[sl], semb))

    lane = lax.iota(jnp.int32, 16)

    def fire_chunk(c):
        par = (c & 1) * _CROWS
        iu = idx_uf[pl.ds(c * _CROWS, _CROWS)]
        ii = idx_if[pl.ds(c * _CROWS, _CROWS)]
        au = lax.shift_right_logical(iu, 3)
        ai = lax.shift_right_logical(ii, 3)
        for j in range(_CROWS):
            s = par + j
            pltpu.async_copy(ue_hbm.at[au[j]], slabs_u.at[s], sem_u)
            pltpu.async_copy(ie_hbm.at[ai[j]], slabs_i.at[s], sem_i)

    def compute_chunk(c):
        par = (c & 1) * _CROWS
        pltpu.make_async_copy(ue_hbm.at[pl.ds(0, _CROWS)],
                              slabs_u.at[pl.ds(par, _CROWS)], sem_u).wait()
        pltpu.make_async_copy(ie_hbm.at[pl.ds(0, _CROWS)],
                              slabs_i.at[pl.ds(par, _CROWS)], sem_i).wait()
        iu = idx_uf[pl.ds(c * _CROWS, _CROWS)]
        ii = idx_if[pl.ds(c * _CROWS, _CROWS)]
        r8u = iu & 7
        r8i = ii & 7
        vecs = []
        for j in range(_CROWS):
            s = par + j
            acc = None
            for t in range(EMBED_DIM // 16):
                tsl = pl.ds(t * 16, 16)
                uv = slabs_u[s, r8u[j], tsl]
                iv = slabs_i[s, r8i[j], tsl]
                acc = uv * iv if acc is None else acc + uv * iv
            vecs.append(acc)
        sh = 1
        while len(vecs) > 1:
            idxs = lane ^ sh
            m = (lane & sh) != 0
            nxt = []
            for q in range(len(vecs) // 2):
                u, v = vecs[2 * q], vecs[2 * q + 1]
                gu = u.at[idxs].get(mode="promise_in_bounds")
                gv = v.at[idxs].get(mode="promise_in_bounds")
                nxt.append(jnp.where(m, v + gv, u + gu))
            vecs = nxt
            sh *= 2
        osl = pl.ds(c * _CROWS, _CROWS)
        out_v[osl] = AVG_RATING + ub_v[osl] + ib_v[osl] + vecs[0]

    def step(c, carry):
        fire_chunk(c)

        @pl.when(c > 0)
        def _():
            compute_chunk(c - 1)

        return carry

    lax.fori_loop(0, _NCC, step, 0)
    for cp in bias_copies:
        cp.wait()
    compute_chunk(_NCC - 1)

    pltpu.sync_copy(out_v, out_hbm.at[pl.ds(base, _BPW)])
    pltpu.sync_copy(ub_v, ub_hbm.at[pl.ds(base, _BPW)])
    pltpu.sync_copy(ib_v, ib_hbm.at[pl.ds(base, _BPW)])


@functools.partial(
    pl.kernel,
    mesh=plsc.VectorSubcoreMesh(core_axis_name="c", subcore_axis_name="s"),
    out_type=(
        jax.ShapeDtypeStruct((BATCH,), jnp.float32),
        jax.ShapeDtypeStruct((BATCH,), jnp.float32),
        jax.ShapeDtypeStruct((BATCH,), jnp.float32),
    ),
    scratch_types=[
        pltpu.VMEM((_NCHUNK, _CHUNK), jnp.int32),
        pltpu.VMEM((_NCHUNK, _CHUNK), jnp.int32),
        pltpu.VMEM((_BPW,), jnp.int32),
        pltpu.VMEM((_BPW,), jnp.int32),
        pltpu.VMEM((2 * _CROWS, 8, EMBED_DIM), jnp.float32),
        pltpu.VMEM((2 * _CROWS, 8, EMBED_DIM), jnp.float32),
        pltpu.VMEM((_BPW,), jnp.float32),
        pltpu.VMEM((_BPW,), jnp.float32),
        pltpu.VMEM((_BPW,), jnp.float32),
        pltpu.SemaphoreType.DMA,
        pltpu.SemaphoreType.DMA,
        pltpu.SemaphoreType.DMA,
    ],
)
def _svd_sc(*refs):
    _body(*refs)


def kernel(user, item, user_emb, item_emb, user_bias, item_bias):
    u3 = user_emb.reshape(user_emb.shape[0] // 8, 8, EMBED_DIM)
    i3 = item_emb.reshape(item_emb.shape[0] // 8, 8, EMBED_DIM)
    return _svd_sc(user, item, u3, i3, user_bias, item_bias)


# speedup vs baseline: 16.9699x; 1.0415x over previous
import functools

import jax
import jax.numpy as jnp
from jax import lax
from jax.experimental import pallas as pl
from jax.experimental.pallas import tpu as pltpu
from jax.experimental.pallas import tpu_sc as plsc

BATCH = 16384
EMBED_DIM = 64
AVG_RATING = 3.0

_NC = 2
_NS = 16
_NW = _NC * _NS
_BPW = BATCH // _NW
_CHUNK = 128
_NCHUNK = _BPW // _CHUNK
_CROWS = 16
_NCC = _BPW // _CROWS
_HALF = _CROWS * 8
_SLOTS = 3


def _body(user_hbm, item_hbm, ue_hbm, ie_hbm,
          user_bias_hbm, item_bias_hbm,
          out_hbm, ub_hbm, ib_hbm,
          idx_u, idx_i, idx_uf, idx_if, slabs_u, slabs_i,
          ub_v, ib_v, out_v, sem_u, sem_i, semb):
    wid = lax.axis_index("s") * _NC + lax.axis_index("c")
    base = wid * _BPW

    for k in range(_NCHUNK):
        src = user_hbm.at[pl.ds(base + k * _CHUNK, _CHUNK)]
        pltpu.sync_copy(src, idx_u.at[k])
        pltpu.sync_copy(src, idx_uf.at[pl.ds(k * _CHUNK, _CHUNK)])
        src = item_hbm.at[pl.ds(base + k * _CHUNK, _CHUNK)]
        pltpu.sync_copy(src, idx_i.at[k])
        pltpu.sync_copy(src, idx_if.at[pl.ds(k * _CHUNK, _CHUNK)])

    bias_copies = []
    for k in range(_NCHUNK):
        sl = pl.ds(k * _CHUNK, _CHUNK)
        bias_copies.append(pltpu.async_copy(user_bias_hbm.at[idx_u.at[k]], ub_v.at[sl], semb))
        bias_copies.append(pltpu.async_copy(item_bias_hbm.at[idx_i.at[k]], ib_v.at[sl], semb))

    lane = lax.iota(jnp.int32, 16)

    def fire_chunk(c):
        slot = lax.rem(c, _SLOTS)
        paru = slot * _CROWS
        pari = slot * _HALF
        iu = idx_uf[pl.ds(c * _CROWS, _CROWS)]
        ii = idx_if[pl.ds(c * _CROWS, _CROWS)]
        au = lax.shift_right_logical(iu, 3)
        ai = ii & (-8)
        for j in range(_CROWS):
            pltpu.async_copy(ue_hbm.at[au[j]], slabs_u.at[paru + j], sem_u)
            si = pl.multiple_of(ai[j], 8)
            pltpu.async_copy(ie_hbm.at[pl.ds(si, 8)],
                             slabs_i.at[pl.ds(pari + j * 8, 8)], sem_i)

    def compute_chunk(c):
        slot = lax.rem(c, _SLOTS)
        paru = slot * _CROWS
        pari = slot * _HALF
        pltpu.make_async_copy(ue_hbm.at[pl.ds(0, _CROWS)],
                              slabs_u.at[pl.ds(paru, _CROWS)], sem_u).wait()
        pltpu.make_async_copy(ie_hbm.at[pl.ds(0, _HALF)],
                              slabs_i.at[pl.ds(pari, _HALF)], sem_i).wait()
        iu = idx_uf[pl.ds(c * _CROWS, _CROWS)]
        ii = idx_if[pl.ds(c * _CROWS, _CROWS)]
        r8u = iu & 7
        r8i = ii & 7
        vecs = []
        for j in range(_CROWS):
            acc = None
            for t in range(EMBED_DIM // 16):
                tsl = pl.ds(t * 16, 16)
                uv = slabs_u[paru + j, r8u[j], tsl]
                iv = slabs_i[pari + j * 8 + r8i[j], tsl]
                acc = uv * iv if acc is None else acc + uv * iv
            vecs.append(acc)
        sh = 1
        while len(vecs) > 1:
            idxs = lane ^ sh
            m = (lane & sh) != 0
            nxt = []
            for q in range(len(vecs) // 2):
                u, v = vecs[2 * q], vecs[2 * q + 1]
                gu = u.at[idxs].get(mode="promise_in_bounds")
                gv = v.at[idxs].get(mode="promise_in_bounds")
                nxt.append(jnp.where(m, v + gv, u + gu))
            vecs = nxt
            sh *= 2
        osl = pl.ds(c * _CROWS, _CROWS)
        out_v[osl] = AVG_RATING + ub_v[osl] + ib_v[osl] + vecs[0]

    fire_chunk(0)
    fire_chunk(1)

    def step(c, carry):
        @pl.when(c + 2 < _NCC)
        def _():
            fire_chunk(c + 2)

        compute_chunk(c)
        return carry

    lax.fori_loop(0, _NCC - 1, step, 0)
    for cp in bias_copies:
        cp.wait()

    def last(c, carry):
        compute_chunk(c)
        return carry

    lax.fori_loop(_NCC - 1, _NCC, last, 0)

    pltpu.sync_copy(out_v, out_hbm.at[pl.ds(base, _BPW)])
    pltpu.sync_copy(ub_v, ub_hbm.at[pl.ds(base, _BPW)])
    pltpu.sync_copy(ib_v, ib_hbm.at[pl.ds(base, _BPW)])


@functools.partial(
    pl.kernel,
    mesh=plsc.VectorSubcoreMesh(core_axis_name="c", subcore_axis_name="s"),
    out_type=(
        jax.ShapeDtypeStruct((BATCH,), jnp.float32),
        jax.ShapeDtypeStruct((BATCH,), jnp.float32),
        jax.ShapeDtypeStruct((BATCH,), jnp.float32),
    ),
    scratch_types=[
        pltpu.VMEM((_NCHUNK, _CHUNK), jnp.int32),
        pltpu.VMEM((_NCHUNK, _CHUNK), jnp.int32),
        pltpu.VMEM((_BPW,), jnp.int32),
        pltpu.VMEM((_BPW,), jnp.int32),
        pltpu.VMEM((_SLOTS * _CROWS, 8, EMBED_DIM), jnp.float32),
        pltpu.VMEM((_SLOTS * _HALF, EMBED_DIM), jnp.float32),
        pltpu.VMEM((_BPW,), jnp.float32),
        pltpu.VMEM((_BPW,), jnp.float32),
        pltpu.VMEM((_BPW,), jnp.float32),
        pltpu.SemaphoreType.DMA,
        pltpu.SemaphoreType.DMA,
        pltpu.SemaphoreType.DMA,
    ],
)
def _svd_sc(*refs):
    _body(*refs)


def kernel(user, item, user_emb, item_emb, user_bias, item_bias):
    u3 = user_emb.reshape(user_emb.shape[0] // 8, 8, EMBED_DIM)
    return _svd_sc(user, item, u3, item_emb, user_bias, item_bias)
